# baseline (speedup 1.0000x reference)
import jax
import jax.numpy as jnp
from jax import lax
from jax.experimental import pallas as pl
from jax.experimental.pallas import tpu as pltpu
from jax.experimental.pallas import tpu_sc as plsc

B = 16384
SEQ = 20
D = 64
H1 = 256
H2 = 128
VOCAB = 1000000

NC, NS, L = 2, 16, 16
NW = NC * NS
BPW = B // NW
C = 16
NCH = BPW // C
CS = SEQ * C
NQ = 4
QS = CS // NQ


def _gather_chunk(emb_hbm, idx_v, rows_v, sem):
    for q in range(NQ):
        pltpu.async_copy(
            emb_hbm.at[idx_v.at[q]], rows_v.at[pl.ds(q * QS, QS)], sem)


def _drain_chunk(emb_hbm, idx_v, rows_v, sem):
    for q in range(NQ):
        pltpu.make_async_copy(
            emb_hbm.at[idx_v.at[q]], rows_v.at[pl.ds(q * QS, QS)], sem).wait()


def _pool_body(idx_hbm, emb_hbm, out_hbm,
               idx0, idx1, rows0, rows1, outb, sem0, sem1):
    wid = lax.axis_index("s") * NC + lax.axis_index("c")
    base = wid * BPW
    wk0 = wid * NCH
    pltpu.sync_copy(idx_hbm.at[wk0], idx0)
    _gather_chunk(emb_hbm, idx0, rows0, sem0)

    def accum(r):
        def acc_body(i, carry):
            for s in range(D // L):
                sl = pl.ds(s * L, L)
                v = r[i * SEQ, sl]
                for j in range(1, SEQ):
                    v = v + r[i * SEQ + j, sl]
                outb[i, sl] = v * (1.0 / SEQ)
            return carry

        lax.fori_loop(0, C, acc_body, 0)

    def pair_body(g, carry):
        k0 = 2 * g
        pltpu.sync_copy(idx_hbm.at[wk0 + k0 + 1], idx1)
        _gather_chunk(emb_hbm, idx1, rows1, sem1)
        _drain_chunk(emb_hbm, idx0, rows0, sem0)
        accum(rows0)
        pltpu.sync_copy(outb, out_hbm.at[pl.ds(base + k0 * C, C)])

        @pl.when(g < NCH // 2 - 1)
        def _():
            pltpu.sync_copy(idx_hbm.at[wk0 + k0 + 2], idx0)
            _gather_chunk(emb_hbm, idx0, rows0, sem0)

        _drain_chunk(emb_hbm, idx1, rows1, sem1)
        accum(rows1)
        pltpu.sync_copy(outb, out_hbm.at[pl.ds(base + (k0 + 1) * C, C)])
        return carry

    lax.fori_loop(0, NCH // 2, pair_body, 0)


_pool = pl.kernel(
    _pool_body,
    out_type=jax.ShapeDtypeStruct((B, D), jnp.float32),
    mesh=plsc.VectorSubcoreMesh(core_axis_name="c", subcore_axis_name="s"),
    scratch_types=[
        pltpu.VMEM((NQ, QS), jnp.int32),
        pltpu.VMEM((NQ, QS), jnp.int32),
        pltpu.VMEM((CS, 2 * D), jnp.float32),
        pltpu.VMEM((CS, 2 * D), jnp.float32),
        pltpu.VMEM((C, D), jnp.float32),
        pltpu.SemaphoreType.DMA,
        pltpu.SemaphoreType.DMA,
    ],
)

TF = 256
NSLAB = VOCAB // TF
TAIL0 = NSLAB * TF
PAIRS = 62


def _fmt_sc_body(embt_hbm, out_hbm, in0, in1, ob0, ob1, semA, semB, semO):
    wid = lax.axis_index("s") * NC + lax.axis_index("c")
    iota = lax.iota(jnp.int32, L)

    def transpose_slab(src, dst):
        for f in range(D):
            for tg in range(TF // L):
                v = src[f, pl.ds(tg * L, L)]
                plsc.store_scatter(dst, [tg * L + iota,
                                         jnp.broadcast_to(f, (L,))], v)

    def in_copy(slab, buf, sem):
        pltpu.async_copy(embt_hbm.at[:, pl.ds(slab * TF, TF)], buf, sem)

    def in_drain(buf, sem):
        pltpu.make_async_copy(embt_hbm.at[:, pl.ds(0, TF)], buf, sem).wait()

    def out_copy(slab, buf):
        pltpu.async_copy(buf, out_hbm.at[pl.ds(slab * TF, TF)], semO)

    def out_drain(buf):
        pltpu.make_async_copy(out_hbm.at[pl.ds(0, TF)], buf, semO).wait()

    in_copy(wid, in0, semA)

    def step(g, carry):
        s0 = wid + (2 * g) * NW
        s1 = s0 + NW

        @pl.when(s1 < NSLAB)
        def _():
            in_copy(s1, in1, semB)

        @pl.when(jnp.logical_and(g > 0, s0 - 2 * NW < NSLAB))
        def _():
            out_drain(ob0)

        @pl.when(jnp.logical_and(g > 0, s1 - 2 * NW < NSLAB))
        def _():
            out_drain(ob1)

        @pl.when(s0 < NSLAB)
        def _():
            in_drain(in0, semA)
            transpose_slab(in0, ob0)
            out_copy(s0, ob0)

        @pl.when(s0 + 2 * NW < NSLAB)
        def _():
            in_copy(s0 + 2 * NW, in0, semA)

        @pl.when(s1 < NSLAB)
        def _():
            in_drain(in1, semB)
            transpose_slab(in1, ob1)
            out_copy(s1, ob1)

        return carry

    lax.fori_loop(0, PAIRS, step, 0)

    @pl.when(wid + (2 * PAIRS - 2) * NW < NSLAB)
    def _():
        out_drain(ob0)

    @pl.when(wid + (2 * PAIRS - 1) * NW < NSLAB)
    def _():
        out_drain(ob1)


_fmt = pl.kernel(
    _fmt_sc_body,
    out_type=jax.ShapeDtypeStruct((VOCAB, 2 * D), jnp.float32),
    mesh=plsc.VectorSubcoreMesh(core_axis_name="c", subcore_axis_name="s"),
    scratch_types=[
        pltpu.VMEM((D, TF), jnp.float32),
        pltpu.VMEM((D, TF), jnp.float32),
        pltpu.VMEM((TF, 2 * D), jnp.float32),
        pltpu.VMEM((TF, 2 * D), jnp.float32),
        pltpu.SemaphoreType.DMA,
        pltpu.SemaphoreType.DMA,
        pltpu.SemaphoreType.DMA,
    ],
    compiler_params=pltpu.CompilerParams(needs_layout_passes=False),
)


def _tail_body(xt_ref, tab_ref, o_ref):
    y = jnp.transpose(xt_ref[...])[:D, :]
    o_ref[...] = jnp.concatenate([y, y], axis=1)


_tail = pl.pallas_call(
    _tail_body,
    grid=(1,),
    in_specs=[
        pl.BlockSpec((D, 2 * D), lambda i: (0, TAIL0 // (2 * D))),
        pl.BlockSpec((D, 2 * D), lambda i: (TAIL0 // D, 0)),
    ],
    out_specs=pl.BlockSpec((D, 2 * D), lambda i: (TAIL0 // D, 0)),
    out_shape=jax.ShapeDtypeStruct((VOCAB, 2 * D), jnp.float32),
    input_output_aliases={1: 0},
)

BLK = 2048


def _mlp_body(x_ref, dh_ref, w1_ref, wdh_ref, b1_ref, w2_ref, b2_ref,
              w3_ref, b3_ref, o_ref):
    h = jnp.dot(x_ref[...], w1_ref[...], preferred_element_type=jnp.float32)
    h = h + jnp.dot(dh_ref[...], wdh_ref[...],
                    preferred_element_type=jnp.float32)
    h = jnp.maximum(h + b1_ref[...][None, :], 0.0)
    h = jnp.maximum(
        jnp.dot(h, w2_ref[...], preferred_element_type=jnp.float32)
        + b2_ref[...][None, :], 0.0)
    o = jnp.dot(h, w3_ref[...], preferred_element_type=jnp.float32)
    o_ref[...] = o + b3_ref[0]


_mlp = pl.pallas_call(
    _mlp_body,
    grid=(B // BLK,),
    in_specs=[
        pl.BlockSpec((BLK, D), lambda i: (i, 0)),
        pl.BlockSpec((BLK, 2), lambda i: (i, 0)),
        pl.BlockSpec((D, H1), lambda i: (0, 0)),
        pl.BlockSpec((2, H1), lambda i: (0, 0)),
        pl.BlockSpec((H1,), lambda i: (0,)),
        pl.BlockSpec((H1, H2), lambda i: (0, 0)),
        pl.BlockSpec((H2,), lambda i: (0,)),
        pl.BlockSpec((H2, 1), lambda i: (0, 0)),
        pl.BlockSpec(memory_space=pltpu.SMEM),
    ],
    out_specs=pl.BlockSpec((BLK, 1), lambda i: (i, 0)),
    out_shape=jax.ShapeDtypeStruct((B, 1), jnp.float32),
)


def kernel(tokenized_titles, day_of_week_num, hour_of_day, emb,
           W1, b1, W2, b2, W3, b3):
    tok = tokenized_titles.astype(jnp.int32)
    idx = tok.reshape(NW * NCH, NQ, QS)
    emb2 = _tail(emb.T, _fmt(emb.T))
    pooled = _pool(idx, emb2)
    dh = jnp.stack([day_of_week_num.astype(jnp.float32),
                    hour_of_day.astype(jnp.float32)], axis=1)
    return _mlp(pooled, dh, W1[:D], W1[D:], b1, W2, b2, W3, b3)[:, 0]


# speedup vs baseline: 3.1408x; 3.1408x over previous
import jax
import jax.numpy as jnp
from jax import lax
from jax.experimental import pallas as pl
from jax.experimental.pallas import tpu as pltpu
from jax.experimental.pallas import tpu_sc as plsc

B = 16384
SEQ = 20
D = 64
H1 = 256
H2 = 128
VOCAB = 1000000

NC, NS, L = 2, 16, 16
NW = NC * NS
BPW = B // NW
C = 16
NCH = BPW // C
CS = SEQ * C
NQ = 4
QS = CS // NQ


def _gather_chunk(emb_hbm, idx_v, rows_v, sem):
    for q in range(NQ):
        pltpu.async_copy(
            emb_hbm.at[idx_v.at[q]], rows_v.at[pl.ds(q * QS, QS)], sem)


def _drain_chunk(emb_hbm, idx_v, rows_v, sem):
    for q in range(NQ):
        pltpu.make_async_copy(
            emb_hbm.at[idx_v.at[q]], rows_v.at[pl.ds(q * QS, QS)], sem).wait()


def _pool_body(idx_hbm, emb_hbm, out_hbm,
               idx0, idx1, rows0, rows1, outb, sem0, sem1):
    wid = lax.axis_index("s") * NC + lax.axis_index("c")
    base = wid * BPW
    wk0 = wid * NCH
    pltpu.sync_copy(idx_hbm.at[wk0], idx0)
    _gather_chunk(emb_hbm, idx0, rows0, sem0)

    def accum(r):
        def acc_body(i, carry):
            for s in range(D // L):
                sl = pl.ds(s * L, L)
                v = r[i * SEQ, sl]
                for j in range(1, SEQ):
                    v = v + r[i * SEQ + j, sl]
                outb[i, sl] = v * (1.0 / SEQ)
            return carry

        lax.fori_loop(0, C, acc_body, 0)

    def pair_body(g, carry):
        k0 = 2 * g
        pltpu.sync_copy(idx_hbm.at[wk0 + k0 + 1], idx1)
        _gather_chunk(emb_hbm, idx1, rows1, sem1)
        _drain_chunk(emb_hbm, idx0, rows0, sem0)
        accum(rows0)
        pltpu.sync_copy(outb, out_hbm.at[pl.ds(base + k0 * C, C)])

        @pl.when(g < NCH // 2 - 1)
        def _():
            pltpu.sync_copy(idx_hbm.at[wk0 + k0 + 2], idx0)
            _gather_chunk(emb_hbm, idx0, rows0, sem0)

        _drain_chunk(emb_hbm, idx1, rows1, sem1)
        accum(rows1)
        pltpu.sync_copy(outb, out_hbm.at[pl.ds(base + (k0 + 1) * C, C)])
        return carry

    lax.fori_loop(0, NCH // 2, pair_body, 0)


_pool = pl.kernel(
    _pool_body,
    out_type=jax.ShapeDtypeStruct((B, D), jnp.float32),
    mesh=plsc.VectorSubcoreMesh(core_axis_name="c", subcore_axis_name="s"),
    scratch_types=[
        pltpu.VMEM((NQ, QS), jnp.int32),
        pltpu.VMEM((NQ, QS), jnp.int32),
        pltpu.VMEM((CS, 2 * D), jnp.float32),
        pltpu.VMEM((CS, 2 * D), jnp.float32),
        pltpu.VMEM((C, D), jnp.float32),
        pltpu.SemaphoreType.DMA,
        pltpu.SemaphoreType.DMA,
    ],
)

TKB = 8192


def _fmt_body(xt_ref, o_ref):
    eye = jnp.eye(D, dtype=jnp.float32)
    y = lax.dot_general(xt_ref[...], eye, (((0,), (0,)), ((), ())),
                        preferred_element_type=jnp.float32)
    o_ref[...] = jnp.concatenate([y, y], axis=1)


_fmt = pl.pallas_call(
    _fmt_body,
    grid=((VOCAB + TKB - 1) // TKB,),
    in_specs=[pl.BlockSpec((D, TKB), lambda i: (0, i))],
    out_specs=pl.BlockSpec((TKB, 2 * D), lambda i: (i, 0)),
    out_shape=jax.ShapeDtypeStruct((VOCAB, 2 * D), jnp.float32),
)

BLK = 2048


def _mlp_body(x_ref, dh_ref, w1_ref, wdh_ref, b1_ref, w2_ref, b2_ref,
              w3_ref, b3_ref, o_ref):
    h = jnp.dot(x_ref[...], w1_ref[...], preferred_element_type=jnp.float32)
    h = h + jnp.dot(dh_ref[...], wdh_ref[...],
                    preferred_element_type=jnp.float32)
    h = jnp.maximum(h + b1_ref[...][None, :], 0.0)
    h = jnp.maximum(
        jnp.dot(h, w2_ref[...], preferred_element_type=jnp.float32)
        + b2_ref[...][None, :], 0.0)
    o = jnp.dot(h, w3_ref[...], preferred_element_type=jnp.float32)
    o_ref[...] = o + b3_ref[0]


_mlp = pl.pallas_call(
    _mlp_body,
    grid=(B // BLK,),
    in_specs=[
        pl.BlockSpec((BLK, D), lambda i: (i, 0)),
        pl.BlockSpec((BLK, 2), lambda i: (i, 0)),
        pl.BlockSpec((D, H1), lambda i: (0, 0)),
        pl.BlockSpec((2, H1), lambda i: (0, 0)),
        pl.BlockSpec((H1,), lambda i: (0,)),
        pl.BlockSpec((H1, H2), lambda i: (0, 0)),
        pl.BlockSpec((H2,), lambda i: (0,)),
        pl.BlockSpec((H2, 1), lambda i: (0, 0)),
        pl.BlockSpec(memory_space=pltpu.SMEM),
    ],
    out_specs=pl.BlockSpec((BLK, 1), lambda i: (i, 0)),
    out_shape=jax.ShapeDtypeStruct((B, 1), jnp.float32),
)


def kernel(tokenized_titles, day_of_week_num, hour_of_day, emb,
           W1, b1, W2, b2, W3, b3):
    tok = tokenized_titles.astype(jnp.int32)
    idx = tok.reshape(NW * NCH, NQ, QS)
    emb2 = _fmt(emb.T)
    pooled = _pool(idx, emb2)
    dh = jnp.stack([day_of_week_num.astype(jnp.float32),
                    hour_of_day.astype(jnp.float32)], axis=1)
    return _mlp(pooled, dh, W1[:D], W1[D:], b1, W2, b2, W3, b3)[:, 0]
